# baseline (speedup 1.0000x reference)
import functools

import jax
import jax.numpy as jnp
from jax import lax
from jax.experimental import pallas as pl
from jax.experimental.pallas import tpu as pltpu
from jax.experimental.pallas import tpu_sc as plsc

_A = 4096
_L = 200
_D = 32
_NW = 32
_ABLK = 128
_LC = 8
_NCHUNK = _L // _LC
_TPC = _LC * _ABLK
_OROWS = _LC * (_D // 8)

_mesh = plsc.VectorSubcoreMesh(core_axis_name="c", subcore_axis_name="s")


@functools.partial(
    pl.kernel,
    mesh=_mesh,
    compiler_params=pltpu.CompilerParams(use_tc_tiling_on_sc=False,
                                         needs_layout_passes=False),
    out_type=jax.ShapeDtypeStruct((_L * (_D // 8), _A // 128, 8 * 128),
                                  jnp.float32),
    scratch_types=[
        pltpu.VMEM((_LC, _ABLK), jnp.int32),
        pltpu.VMEM((_LC, _ABLK), jnp.int32),
        pltpu.VMEM((_TPC, _D), jnp.float32),
        pltpu.VMEM((_TPC, _D), jnp.float32),
        pltpu.VMEM((_OROWS * 8 * _ABLK,), jnp.float32),
        pltpu.VMEM((_L, _D), jnp.float32),
        pltpu.SemaphoreType.DMA,
        pltpu.SemaphoreType.DMA,
        pltpu.SemaphoreType.DMA,
        pltpu.SemaphoreType.DMA,
        pltpu.SemaphoreType.DMA,
    ],
)
def _sc_embed(xt_ref, tok_ref, pos_ref, out_ref,
              idx0, idx1, rows0, rows1, outv, pos_v,
              si0, si1, sg0, sg1, so):
    wid = lax.axis_index("s") * 2 + lax.axis_index("c")
    a0 = wid * _ABLK

    idxb = (idx0, idx1)
    rowsb = (rows0, rows1)
    si = (si0, si1)
    sg = (sg0, sg1)

    pltpu.sync_copy(pos_ref, pos_v)

    e = lax.iota(jnp.int32, 16)
    pat = lax.div(e, 8) * (8 * _ABLK) + lax.rem(e, 8) * _ABLK

    def start_idx(c, b):
        return pltpu.async_copy(
            xt_ref.at[pl.ds(c * _LC, _LC), pl.ds(a0, _ABLK)],
            idxb[b], si[b])

    def start_gathers(b):
        return [
            pltpu.async_copy(
                tok_ref.at[idxb[b].at[lr]],
                rowsb[b].at[pl.ds(lr * _ABLK, _ABLK)],
                sg[b])
            for lr in range(_LC)
        ]

    def start_out(c):
        return [
            pltpu.async_copy(
                outv.at[pl.ds(row * 8 * _ABLK, 8 * _ABLK)],
                out_ref.at[c * _OROWS + row, wid],
                so)
            for row in range(_OROWS)
        ]

    def tec_pass(c, b):
        rb = rowsb[b]
        l0 = c * _LC

        def lbody(lrel, carry):
            p0 = pos_v[l0 + lrel, pl.ds(0, 16)]
            p1 = pos_v[l0 + lrel, pl.ds(16, 16)]
            pat0 = pat + lrel * (4 * 8 * _ABLK)
            pat1 = pat0 + 2 * 8 * _ABLK

            def abody(a, carry2):
                t = lrel * _ABLK + a
                off0 = pat0 + a
                off1 = pat1 + a
                v0 = rb[t, pl.ds(0, 16)] + p0
                plsc.store_scatter(outv, [off0], v0)
                v1 = rb[t, pl.ds(16, 16)] + p1
                plsc.store_scatter(outv, [off1], v1)
                return carry2

            lax.fori_loop(0, _ABLK, abody, 0)
            return carry

        lax.fori_loop(0, _LC, lbody, 0)

    cpi = [start_idx(0, 0), start_idx(1, 1)]
    cpi[0].wait()
    cpg = [start_gathers(0), None]
    cpo = None
    for c in range(_NCHUNK):
        b = c & 1
        nb = 1 - b
        for cp in cpg[b]:
            cp.wait()
        if c + 2 < _NCHUNK:
            cpi[b] = start_idx(c + 2, b)
        if c + 1 < _NCHUNK:
            cpi[nb].wait()
            cpg[nb] = start_gathers(nb)
        if cpo is not None:
            for cp in cpo:
                cp.wait()
        tec_pass(c, b)
        cpo = start_out(c)
    for cp in cpo:
        cp.wait()


def kernel(x, token_table, pos_table):
    xt = x.T.astype(jnp.int32)
    out3 = _sc_embed(xt, token_table, pos_table)
    out5 = out3.reshape(_L, _D // 8, _A // 128, 8, 128)
    o = out5.transpose(2, 4, 0, 1, 3)
    return o.reshape(_A, _L, _D)


# speedup vs baseline: 1.0074x; 1.0074x over previous
import functools

import jax
import jax.numpy as jnp
from jax import lax
from jax.experimental import pallas as pl
from jax.experimental.pallas import tpu as pltpu
from jax.experimental.pallas import tpu_sc as plsc

_A = 4096
_L = 200
_D = 32
_NPG = 8
_NQ = 4
_LPW = _L // _NPG
_ABLK = _A // _NQ
_MW = _ABLK // 128

_mesh = plsc.VectorSubcoreMesh(core_axis_name="c", subcore_axis_name="s")


@functools.partial(
    pl.kernel,
    mesh=_mesh,
    compiler_params=pltpu.CompilerParams(use_tc_tiling_on_sc=False,
                                         needs_layout_passes=False),
    out_type=jax.ShapeDtypeStruct((_L * (_D // 8), _A // 128, 8 * 128),
                                  jnp.float32),
    scratch_types=[
        pltpu.VMEM((_ABLK,), jnp.int32),
        pltpu.VMEM((_ABLK,), jnp.int32),
        pltpu.VMEM((_ABLK, _D), jnp.float32),
        pltpu.VMEM((_ABLK, _D), jnp.float32),
        pltpu.VMEM((_D // 8, _MW, 8 * 128), jnp.float32),
        pltpu.VMEM((_LPW, _D), jnp.float32),
        pltpu.SemaphoreType.DMA,
        pltpu.SemaphoreType.DMA,
        pltpu.SemaphoreType.DMA,
        pltpu.SemaphoreType.DMA,
        pltpu.SemaphoreType.DMA,
    ],
)
def _sc_embed(xt_ref, tok_ref, pos_ref, out_ref,
              idx0, idx1, rows0, rows1, outv, pos_v,
              si0, si1, sg0, sg1, so):
    wid = lax.axis_index("s") * 2 + lax.axis_index("c")
    pg = lax.div(wid, _NQ)
    q = lax.rem(wid, _NQ)
    l0 = pg * _LPW
    a0 = q * _ABLK

    idxb = (idx0, idx1)
    rowsb = (rows0, rows1)
    si = (si0, si1)
    sg = (sg0, sg1)

    pltpu.sync_copy(pos_ref.at[pl.ds(l0, _LPW)], pos_v)

    e = lax.iota(jnp.int32, 16)
    rowpat = lax.div(e, 8)
    colpat = lax.rem(e, 8) * 128

    def start_idx(c, b):
        return pltpu.async_copy(
            xt_ref.at[l0 + c, pl.ds(a0, _ABLK)], idxb[b], si[b])

    def start_gather(b):
        return pltpu.async_copy(
            tok_ref.at[idxb[b]], rowsb[b], sg[b])

    def start_out(c):
        return pltpu.async_copy(
            outv,
            out_ref.at[pl.ds((l0 + c) * (_D // 8), _D // 8),
                       pl.ds(q * _MW, _MW)],
            so)

    def tec_pass(c, b):
        rb = rowsb[b]
        p0 = pos_v[c, pl.ds(0, 16)]
        p1 = pos_v[c, pl.ds(16, 16)]
        row1 = rowpat + 2

        def mbody(m, carry):
            mv = jnp.broadcast_to(m, (16,))

            def abody(a, carry2):
                t = m * 128 + a
                col = colpat + a
                v0 = rb[t, pl.ds(0, 16)] + p0
                plsc.store_scatter(outv, [rowpat, mv, col], v0)
                v1 = rb[t, pl.ds(16, 16)] + p1
                plsc.store_scatter(outv, [row1, mv, col], v1)
                return carry2

            lax.fori_loop(0, 128, abody, 0)
            return carry

        lax.fori_loop(0, _MW, mbody, 0)

    cpi = [start_idx(0, 0), start_idx(1, 1)]
    cpi[0].wait()
    cpg = [start_gather(0), None]
    cpo = None
    for c in range(_LPW):
        b = c & 1
        nb = 1 - b
        cpg[b].wait()
        if c + 2 < _LPW:
            cpi[b] = start_idx(c + 2, b)
        if c + 1 < _LPW:
            cpi[nb].wait()
            cpg[nb] = start_gather(nb)
        if cpo is not None:
            cpo.wait()
        tec_pass(c, b)
        cpo = start_out(c)
    cpo.wait()


def kernel(x, token_table, pos_table):
    xt = x.T.astype(jnp.int32)
    out3 = _sc_embed(xt, token_table, pos_table)
    out5 = out3.reshape(_L, _D // 8, _A // 128, 8, 128)
    o = out5.transpose(2, 4, 0, 1, 3)
    return o.reshape(_A, _L, _D)


# speedup vs baseline: 1.1416x; 1.1332x over previous
import functools

import jax
import jax.numpy as jnp
from jax import lax
from jax.experimental import pallas as pl
from jax.experimental.pallas import tpu as pltpu
from jax.experimental.pallas import tpu_sc as plsc

_B = 4096
_L = 200
_D = 32
_FLAT = _B * _L
_NW = 32
_PER_W = _FLAT // _NW
_C = 1600
_CHUNKS = _PER_W // _C
_PERIODS = _C // _L

_SUBS = [(0, _C)]

_mesh = plsc.VectorSubcoreMesh(core_axis_name="c", subcore_axis_name="s")


@functools.partial(
    pl.kernel,
    mesh=_mesh,
    compiler_params=pltpu.CompilerParams(use_tc_tiling_on_sc=False),
    out_type=jax.ShapeDtypeStruct((_FLAT, _D), jnp.float32),
    scratch_types=[
        pltpu.VMEM((_C,), jnp.int32),
        pltpu.VMEM((_C,), jnp.int32),
        pltpu.VMEM((_C, _D), jnp.float32),
        pltpu.VMEM((_C, _D), jnp.float32),
        pltpu.VMEM((_L, _D), jnp.float32),
        pltpu.SemaphoreType.DMA,
        pltpu.SemaphoreType.DMA,
        pltpu.SemaphoreType.DMA,
        pltpu.SemaphoreType.DMA,
        pltpu.SemaphoreType.DMA,
        pltpu.SemaphoreType.DMA,
    ],
)
def _sc_embed(x_ref, tok_ref, pos_ref, out_ref,
              idx0, idx1, rows0, rows1, pos_v,
              si0, si1, sg0, sg1, so0, so1):
    wid = lax.axis_index("s") * 2 + lax.axis_index("c")
    base = wid * _PER_W

    idxb = (idx0, idx1)
    rowsb = (rows0, rows1)
    si = (si0, si1)
    sg = (sg0, sg1)
    so = (so0, so1)

    pltpu.sync_copy(pos_ref, pos_v)

    def start_idx(g, b):
        return pltpu.async_copy(
            x_ref.at[pl.ds(base + g * _C, _C)], idxb[b], si[b])

    def start_gathers(b):
        return [
            pltpu.async_copy(
                tok_ref.at[idxb[b].at[pl.ds(s, n)]],
                rowsb[b].at[pl.ds(s, n)],
                sg[b])
            for (s, n) in _SUBS
        ]

    def start_scat(g, b):
        return pltpu.async_copy(
            rowsb[b], out_ref.at[pl.ds(base + g * _C, _C)], so[b])

    def do_add(b):
        rb = rowsb[b]

        def lbody(l, carry):
            p0 = pos_v[l, pl.ds(0, 16)]
            p1 = pos_v[l, pl.ds(16, 16)]
            for k in range(_PERIODS):
                r = l + _L * k
                rb[r, pl.ds(0, 16)] += p0
                rb[r, pl.ds(16, 16)] += p1
            return carry

        lax.fori_loop(0, _L, lbody, 0)

    cpi = [start_idx(0, 0), start_idx(1, 1)]
    cpi[0].wait()
    cpg = [start_gathers(0), None]
    cpo = [None, None]
    for g in range(_CHUNKS):
        b = g & 1
        nb = 1 - b
        for c in cpg[b]:
            c.wait()
        if g + 2 < _CHUNKS:
            cpi[b] = start_idx(g + 2, b)
        if g + 1 < _CHUNKS:
            cpi[nb].wait()
            if cpo[nb] is not None:
                cpo[nb].wait()
            cpg[nb] = start_gathers(nb)
        do_add(b)
        cpo[b] = start_scat(g, b)
    cpo[0].wait()
    cpo[1].wait()


def kernel(x, token_table, pos_table):
    xf = x.reshape(-1).astype(jnp.int32)
    out = _sc_embed(xf, token_table, pos_table)
    return out.reshape(_B, _L, _D)


# speedup vs baseline: 1.1420x; 1.0003x over previous
import functools

import jax
import jax.numpy as jnp
from jax import lax
from jax.experimental import pallas as pl
from jax.experimental.pallas import tpu as pltpu
from jax.experimental.pallas import tpu_sc as plsc

_B = 4096
_L = 200
_D = 32
_NW = 32
_BPW = _B // _NW
_RB = 8
_CHUNKS = _BPW // _RB

_mesh = plsc.VectorSubcoreMesh(core_axis_name="c", subcore_axis_name="s")


@functools.partial(
    pl.kernel,
    mesh=_mesh,
    compiler_params=pltpu.CompilerParams(use_tc_tiling_on_sc=False),
    out_type=jax.ShapeDtypeStruct((_B, _L, _D), jnp.float32),
    scratch_types=[
        pltpu.VMEM((_RB, _L), jnp.int32),
        pltpu.VMEM((_RB, _L), jnp.int32),
        pltpu.VMEM((_RB, _L, _D), jnp.float32),
        pltpu.VMEM((_RB, _L, _D), jnp.float32),
        pltpu.VMEM((_L, _D), jnp.float32),
        pltpu.SemaphoreType.DMA,
        pltpu.SemaphoreType.DMA,
        pltpu.SemaphoreType.DMA,
        pltpu.SemaphoreType.DMA,
        pltpu.SemaphoreType.DMA,
        pltpu.SemaphoreType.DMA,
    ],
)
def _sc_embed(x_ref, tok_ref, pos_ref, out_ref,
              idx0, idx1, rows0, rows1, pos_v,
              si0, si1, sg0, sg1, so0, so1):
    wid = lax.axis_index("s") * 2 + lax.axis_index("c")
    b0 = wid * _BPW

    idxb = (idx0, idx1)
    rowsb = (rows0, rows1)
    si = (si0, si1)
    sg = (sg0, sg1)
    so = (so0, so1)

    pltpu.sync_copy(pos_ref, pos_v)

    def start_idx(g, b):
        return pltpu.async_copy(
            x_ref.at[pl.ds(b0 + g * _RB, _RB)], idxb[b], si[b])

    def start_gathers(b):
        return [
            pltpu.async_copy(
                tok_ref.at[idxb[b].at[r]], rowsb[b].at[r], sg[b])
            for r in range(_RB)
        ]

    def start_scat(g, b):
        return pltpu.async_copy(
            rowsb[b], out_ref.at[pl.ds(b0 + g * _RB, _RB)], so[b])

    def do_add(b):
        rb = rowsb[b]

        def lbody(l, carry):
            p0 = pos_v[l, pl.ds(0, 16)]
            p1 = pos_v[l, pl.ds(16, 16)]
            for r in range(_RB):
                rb[r, l, pl.ds(0, 16)] += p0
                rb[r, l, pl.ds(16, 16)] += p1
            return carry

        lax.fori_loop(0, _L, lbody, 0)

    cpi = [start_idx(0, 0), start_idx(1, 1)]
    cpi[0].wait()
    cpg = [start_gathers(0), None]
    cpo = [None, None]
    for g in range(_CHUNKS):
        b = g & 1
        nb = 1 - b
        for c in cpg[b]:
            c.wait()
        if g + 2 < _CHUNKS:
            cpi[b] = start_idx(g + 2, b)
        if g + 1 < _CHUNKS:
            cpi[nb].wait()
            if cpo[nb] is not None:
                cpo[nb].wait()
            cpg[nb] = start_gathers(nb)
        do_add(b)
        cpo[b] = start_scat(g, b)
    cpo[0].wait()
    cpo[1].wait()


def kernel(x, token_table, pos_table):
    return _sc_embed(x.astype(jnp.int32), token_table, pos_table)
